# initial kernel scaffold (unmeasured)
import jax
import jax.numpy as jnp
from jax import lax
from jax.experimental import pallas as pl
from jax.experimental.pallas import tpu as pltpu

N_DEV = 8
M_PER = 512
N_PER = 256
K = 4096
N_TOT = 2048


def kernel(x, w_mat, scale_x, scale_w):
    def body(x_ref, w_ref, sx_ref, sw_ref, out_ref,
             chunks, send_sems, recv_sems, copy_sem):
        me = lax.axis_index("i")

        acc = jnp.dot(x_ref[:, :], w_ref[:, :],
                      preferred_element_type=jnp.int32)
        scale = sx_ref[0] * sw_ref[0]
        y = acc.astype(jnp.float32) * scale
        yc = jnp.clip(y, -60.0, 60.0)
        y = y / (1.0 + jnp.exp(-yc))

        for d in range(N_DEV):
            chunks[d, :, :] = y[:, d * N_PER:(d + 1) * N_PER]

        my_rows = pl.ds(me * M_PER, M_PER)

        local_copy = pltpu.make_async_copy(
            chunks.at[me], out_ref.at[my_rows, :], copy_sem)
        local_copy.start()

        rdmas = []
        for o in range(1, N_DEV):
            dst = (me + o) % N_DEV
            rdma = pltpu.make_async_remote_copy(
                src_ref=chunks.at[dst],
                dst_ref=out_ref.at[my_rows, :],
                send_sem=send_sems.at[o],
                recv_sem=recv_sems.at[o],
                device_id=(dst,),
                device_id_type=pl.DeviceIdType.MESH,
            )
            rdma.start()
            rdmas.append(rdma)

        local_copy.wait()
        for rdma in rdmas:
            rdma.wait_send()

        for o in range(1, N_DEV):
            src = (me - o) % N_DEV
            recv = pltpu.make_async_remote_copy(
                src_ref=chunks.at[0],
                dst_ref=out_ref.at[pl.ds(src * M_PER, M_PER), :],
                send_sem=send_sems.at[o],
                recv_sem=recv_sems.at[o],
                device_id=(me,),
                device_id_type=pl.DeviceIdType.MESH,
            )
            recv.wait_recv()

    return pl.pallas_call(
        body,
        out_shape=jax.ShapeDtypeStruct((N_DEV * M_PER, N_PER), jnp.float32),
        in_specs=[
            pl.BlockSpec(memory_space=pltpu.VMEM),
            pl.BlockSpec(memory_space=pltpu.VMEM),
            pl.BlockSpec(memory_space=pltpu.SMEM),
            pl.BlockSpec(memory_space=pltpu.SMEM),
        ],
        out_specs=pl.BlockSpec(memory_space=pltpu.VMEM),
        scratch_shapes=[
            pltpu.VMEM((N_DEV, M_PER, N_PER), jnp.float32),
            pltpu.SemaphoreType.DMA((N_DEV,)),
            pltpu.SemaphoreType.DMA((N_DEV,)),
            pltpu.SemaphoreType.DMA,
        ],
        compiler_params=pltpu.CompilerParams(collective_id=0),
    )(x, w_mat, scale_x, scale_w)


# baseline (device time: 57450 ns/iter reference)
import jax
import jax.numpy as jnp
from jax import lax
from jax.experimental import pallas as pl
from jax.experimental.pallas import tpu as pltpu

N_DEV = 8
M_PER = 512
N_PER = 256
K = 4096
N_TOT = 2048


def kernel(x, w_mat, scale_x, scale_w):
    def body(x_ref, w_ref, sx_ref, sw_ref, out_ref,
             chunks, send_sems, recv_sems, copy_sem):
        me = lax.axis_index("i")

        acc = jnp.dot(x_ref[:, :], w_ref[:, :],
                      preferred_element_type=jnp.int32)
        scale = sx_ref[0] * sw_ref[0]
        y = acc.astype(jnp.float32) * scale
        yc = jnp.clip(y, -60.0, 60.0)
        y = y / (1.0 + jnp.exp(-yc))

        for d in range(N_DEV):
            chunks[d, :, :] = y[:, d * N_PER:(d + 1) * N_PER]

        my_rows = pl.ds(me * M_PER, M_PER)

        local_copy = pltpu.make_async_copy(
            chunks.at[me], out_ref.at[my_rows, :], copy_sem)
        local_copy.start()

        rdmas = []
        for o in range(1, N_DEV):
            dst = (me + o) % N_DEV
            rdma = pltpu.make_async_remote_copy(
                src_ref=chunks.at[dst],
                dst_ref=out_ref.at[my_rows, :],
                send_sem=send_sems.at[o],
                recv_sem=recv_sems.at[o],
                device_id=(dst,),
                device_id_type=pl.DeviceIdType.MESH,
            )
            rdma.start()
            rdmas.append(rdma)

        local_copy.wait()
        for rdma in rdmas:
            rdma.wait_send()

        for o in range(1, N_DEV):
            src = (me - o) % N_DEV
            recv = pltpu.make_async_remote_copy(
                src_ref=chunks.at[0],
                dst_ref=out_ref.at[pl.ds(src * M_PER, M_PER), :],
                send_sem=send_sems.at[o],
                recv_sem=recv_sems.at[o],
                device_id=(me,),
                device_id_type=pl.DeviceIdType.MESH,
            )
            recv.wait_recv()

    return pl.pallas_call(
        body,
        out_shape=jax.ShapeDtypeStruct((N_DEV * M_PER, N_PER), jnp.float32),
        in_specs=[
            pl.BlockSpec(memory_space=pltpu.VMEM),
            pl.BlockSpec(memory_space=pltpu.VMEM),
            pl.BlockSpec(memory_space=pltpu.SMEM),
            pl.BlockSpec(memory_space=pltpu.SMEM),
        ],
        out_specs=pl.BlockSpec(memory_space=pltpu.VMEM),
        scratch_shapes=[
            pltpu.VMEM((N_DEV, M_PER, N_PER), jnp.float32),
            pltpu.SemaphoreType.DMA((N_DEV,)),
            pltpu.SemaphoreType.DMA((N_DEV,)),
            pltpu.SemaphoreType.DMA,
        ],
    )(x, w_mat, scale_x, scale_w)
